# baseline (speedup 1.0000x reference)
import functools

import jax
import jax.numpy as jnp
from jax import lax
from jax.experimental import pallas as pl
from jax.experimental.pallas import tpu as pltpu
from jax.experimental.pallas import tpu_sc as plsc

N = 10000
NP = 10240
H = 128
E = 320000
WIN = 128
WP = 2560
EP = WP * WIN
NCORE = 2
NSUB = 16
WPS = WP // NSUB
KW = 8
RPS = NP // NSUB
B = 1024
G = NP // B

_f32 = jnp.float32


def _dot(a, b):
    return jnp.dot(a, b, preferred_element_type=_f32,
                   precision=lax.Precision.HIGHEST)


def _ln(u, g, b):
    mu = jnp.mean(u, axis=-1, keepdims=True)
    var = jnp.mean((u - mu) ** 2, axis=-1, keepdims=True)
    return (u - mu) / jnp.sqrt(var + 1e-5) * g + b


def _split_agg(a0, a1):
    acc = jnp.concatenate([a0[:, :64], a1[:, :64]], axis=1)
    den = jnp.concatenate([a0[:, 64:], a1[:, 64:]], axis=1)
    return acc, den


def _fc_body(x_ref, w_ref, b_ref, h_ref, m_ref):
    h = jnp.maximum(_dot(x_ref[...], w_ref[...]) + b_ref[...], 0.0)
    h_ref[...] = h
    m_ref[...] = h + 1e-7


def _conv0_body(acc_ref, den_ref, h_ref, w1_ref, b1_ref, g1_ref,
                be1_ref, w2_ref, b2_ref, ho_ref, m_ref):
    acc, den = _split_agg(acc_ref[0], den_ref[0])
    agg = acc / (den + 1e-16) + h_ref[...]
    u = _dot(agg, w1_ref[...]) + b1_ref[...]
    u = jnp.maximum(_ln(u, g1_ref[...], be1_ref[...]), 0.0)
    h1 = _dot(u, w2_ref[...]) + b2_ref[...]
    ho_ref[...] = h1
    m_ref[...] = jnp.maximum(h1, 0.0) + 1e-7


def _convmid_body(acc_ref, den_ref, h_ref, w1_ref, b1_ref, g1_ref,
                  be1_ref, w2_ref, b2_ref, ng_ref, nb_ref, ho_ref, m_ref):
    acc, den = _split_agg(acc_ref[0], den_ref[0])
    agg = acc / (den + 1e-16) + h_ref[...]
    u = _dot(agg, w1_ref[...]) + b1_ref[...]
    u = jnp.maximum(_ln(u, g1_ref[...], be1_ref[...]), 0.0)
    c = _dot(u, w2_ref[...]) + b2_ref[...]
    c = jnp.maximum(_ln(c, ng_ref[...], nb_ref[...]), 0.0)
    hn = h_ref[...] + c
    ho_ref[...] = hn
    m_ref[...] = jnp.maximum(hn, 0.0) + 1e-7


def _convlast_body(acc_ref, den_ref, h_ref, w1_ref, b1_ref, g1_ref,
                   be1_ref, w2_ref, b2_ref, ng_ref, nb_ref, ho_ref):
    acc, den = _split_agg(acc_ref[0], den_ref[0])
    agg = acc / (den + 1e-16) + h_ref[...]
    u = _dot(agg, w1_ref[...]) + b1_ref[...]
    u = jnp.maximum(_ln(u, g1_ref[...], be1_ref[...]), 0.0)
    c = _dot(u, w2_ref[...]) + b2_ref[...]
    c = jnp.maximum(_ln(c, ng_ref[...], nb_ref[...]), 0.0)
    ho_ref[...] = h_ref[...] + c


def _attn_body(cb_ref, h0_ref, h1_ref, h2_ref, h3_ref, pw_ref, pb_ref,
               aw_ref, ab_ref, bw_ref, bb_ref, cw_ref, hp_ref, a_ref):
    hcat = jnp.concatenate(
        [h0_ref[...], h1_ref[...], h2_ref[...], h3_ref[...]], axis=1)
    hp = jnp.maximum(_dot(hcat, pw_ref[...]) + pb_ref[...], 0.0)
    a = jnp.tanh(_dot(hp, aw_ref[...]) + ab_ref[...])
    g = 1.0 / (1.0 + jnp.exp(-(_dot(hp, bw_ref[...]) + bb_ref[...])))
    hp_ref[...] = hp
    a_ref[...] = jnp.sum(a * g * cw_ref[...], axis=1,
                         keepdims=True) + cb_ref[0, 0]


def _pool_body(a_ref, hp_ref, rw_ref, rb_ref, cw_ref, cb_ref, o_ref):
    mask = lax.broadcasted_iota(jnp.int32, (NP, 1), 0) < N
    am = jnp.where(mask, a_ref[...], -jnp.inf)
    mx = jnp.max(am)
    e = jnp.exp(am - mx)
    w = e / jnp.sum(e)
    hp = jnp.where(mask, hp_ref[...], 0.0)
    pooled = jnp.sum(w * hp, axis=0, keepdims=True)
    hr = jnp.maximum(_dot(pooled, rw_ref[...]) + rb_ref[...], 0.0)
    logits = _dot(hr, cw_ref[...]) + cb_ref[...]
    o_ref[...] = 1.0 / (1.0 + jnp.exp(-logits))


def _row(x):
    return pl.BlockSpec((B, x), lambda i: (i, 0))


def _w(shape):
    return pl.BlockSpec(shape, lambda i: (0, 0))


_SM = pl.BlockSpec(memory_space=pltpu.SMEM)

_fc_call = pl.pallas_call(
    _fc_body, grid=(G,),
    in_specs=[_row(H), _w((H, H)), _w((1, H))],
    out_specs=[_row(H), _row(H)],
    out_shape=[jax.ShapeDtypeStruct((NP, H), _f32),
               jax.ShapeDtypeStruct((NP, H), _f32)],
)

_agg0 = pl.BlockSpec((1, B, H), lambda i: (0, i, 0))
_agg1 = pl.BlockSpec((1, B, H), lambda i: (1, i, 0))

_conv0_call = pl.pallas_call(
    _conv0_body, grid=(G,),
    in_specs=[_agg0, _agg1, _row(H), _w((H, 2 * H)), _w((1, 2 * H)),
              _w((1, 2 * H)), _w((1, 2 * H)), _w((2 * H, H)), _w((1, H))],
    out_specs=[_row(H), _row(H)],
    out_shape=[jax.ShapeDtypeStruct((NP, H), _f32),
               jax.ShapeDtypeStruct((NP, H), _f32)],
)

_convmid_call = pl.pallas_call(
    _convmid_body, grid=(G,),
    in_specs=[_agg0, _agg1, _row(H), _w((H, 2 * H)), _w((1, 2 * H)),
              _w((1, 2 * H)), _w((1, 2 * H)), _w((2 * H, H)), _w((1, H)),
              _w((1, H)), _w((1, H))],
    out_specs=[_row(H), _row(H)],
    out_shape=[jax.ShapeDtypeStruct((NP, H), _f32),
               jax.ShapeDtypeStruct((NP, H), _f32)],
)

_convlast_call = pl.pallas_call(
    _convlast_body, grid=(G,),
    in_specs=[_agg0, _agg1, _row(H), _w((H, 2 * H)), _w((1, 2 * H)),
              _w((1, 2 * H)), _w((1, 2 * H)), _w((2 * H, H)), _w((1, H)),
              _w((1, H)), _w((1, H))],
    out_specs=_row(H),
    out_shape=jax.ShapeDtypeStruct((NP, H), _f32),
)

_attn_call = pl.pallas_call(
    _attn_body, grid=(G,),
    in_specs=[_SM, _row(H), _row(H), _row(H), _row(H),
              _w((4 * H, 4 * H)), _w((1, 4 * H)),
              _w((4 * H, 4 * H)), _w((1, 4 * H)),
              _w((4 * H, 4 * H)), _w((1, 4 * H)), _w((1, 4 * H))],
    out_specs=[_row(4 * H), pl.BlockSpec((B, 1), lambda i: (i, 0))],
    out_shape=[jax.ShapeDtypeStruct((NP, 4 * H), _f32),
               jax.ShapeDtypeStruct((NP, 1), _f32)],
)

_pool_call = pl.pallas_call(
    _pool_body,
    in_specs=[pl.BlockSpec((NP, 1), lambda: (0, 0)),
              pl.BlockSpec((NP, 4 * H), lambda: (0, 0)),
              pl.BlockSpec((4 * H, 4 * H), lambda: (0, 0)),
              pl.BlockSpec((1, 4 * H), lambda: (0, 0)),
              pl.BlockSpec((4 * H, 2), lambda: (0, 0)),
              pl.BlockSpec((1, 2), lambda: (0, 0))],
    out_specs=pl.BlockSpec((1, 2), lambda: (0, 0)),
    out_shape=jax.ShapeDtypeStruct((1, 2), _f32),
)


KG = 16
NCHUNK = WPS // KG
GPC = KG // 2


def _sc_agg_body(msg_ref, idxc_ref, tv_ref, out_ref,
                 mb0, mb1, sbuf, ib0, ib1, tvm, spacc,
                 g0, g1, s0, i0, i1):
    c = lax.axis_index("c")
    s = lax.axis_index("s")

    @pl.loop(0, WIN)
    def _zrow(i):
        @pl.loop(0, H // 16)
        def _zcol(j):
            sbuf[i, pl.ds(j * 16, 16)] = jnp.zeros((16,), _f32)

    @pl.loop(0, RPS // WIN)
    def _zacc(k):
        pltpu.sync_copy(sbuf, spacc.at[pl.ds(s * RPS + k * WIN, WIN)])

    pltpu.sync_copy(tv_ref, tvm)
    tv = tvm[...]

    def idx_load(cc, ib, isem):
        pltpu.async_copy(
            idxc_ref.at[c, pl.ds(s * WPS + cc * KG, KG)], ib, isem)

    def compute(mb):
        @plsc.parallel_loop(0, WIN, unroll=16)
        def _e(e):
            for j in range(4):
                m = mb[e, pl.ds(j * 16, 16)]
                q = jnp.exp(m * tv)
                sbuf[e, pl.ds(64 + j * 16, 16)] = q
                sbuf[e, pl.ds(j * 16, 16)] = m * q

    def do_chunk(ib):
        pltpu.async_copy(msg_ref.at[ib.at[0, 0]], mb0, g0)
        pltpu.async_copy(msg_ref.at[ib.at[1, 0]], mb1, g1)

        @pl.loop(0, GPC)
        def _pair(gg):
            w = gg * 2
            pltpu.make_async_copy(msg_ref.at[ib.at[w, 0]], mb0, g0).wait()
            compute(mb0)
            pltpu.async_copy(sbuf, spacc.at[ib.at[w, 1]], s0, add=True)

            @pl.when(gg < GPC - 1)
            def _f0():
                pltpu.async_copy(msg_ref.at[ib.at[w + 2, 0]], mb0, g0)

            pltpu.make_async_copy(msg_ref.at[ib.at[w + 1, 0]], mb1, g1).wait()
            pltpu.make_async_copy(sbuf, spacc.at[ib.at[w, 1]], s0).wait()
            compute(mb1)
            pltpu.async_copy(sbuf, spacc.at[ib.at[w + 1, 1]], s0, add=True)

            @pl.when(gg < GPC - 1)
            def _f1():
                pltpu.async_copy(msg_ref.at[ib.at[w + 3, 0]], mb1, g1)

            pltpu.make_async_copy(sbuf, spacc.at[ib.at[w + 1, 1]], s0).wait()

    idx_load(0, ib0, i0)
    plsc.subcore_barrier()

    @pl.loop(0, NCHUNK // 2)
    def _chunkpair(k):
        cc = k * 2
        pltpu.make_async_copy(
            idxc_ref.at[c, pl.ds(s * WPS + cc * KG, KG)], ib0, i0).wait()
        idx_load(cc + 1, ib1, i1)
        do_chunk(ib0)
        pltpu.make_async_copy(
            idxc_ref.at[c, pl.ds(s * WPS + (cc + 1) * KG, KG)], ib1, i1).wait()

        @pl.when(k < NCHUNK // 2 - 1)
        def _nxt():
            idx_load(cc + 2, ib0, i0)

        do_chunk(ib1)

    plsc.subcore_barrier()
    pltpu.sync_copy(spacc.at[pl.ds(s * RPS, RPS)],
                    out_ref.at[c, pl.ds(s * RPS, RPS)])


@functools.cache
def _sc_agg_call():
    return pl.kernel(
        _sc_agg_body,
        out_type=jax.ShapeDtypeStruct((NCORE, NP, H), _f32),
        mesh=plsc.VectorSubcoreMesh(core_axis_name="c", subcore_axis_name="s"),
        compiler_params=pltpu.CompilerParams(use_tc_tiling_on_sc=False),
        scratch_types=(
            [pltpu.VMEM((WIN, H // 2), _f32) for _ in range(2)]
            + [pltpu.VMEM((WIN, H), _f32)]
            + [pltpu.VMEM((KG, 2, WIN), jnp.int32) for _ in range(2)]
            + [pltpu.VMEM((16,), _f32)]
            + [pltpu.VMEM_SHARED((NP, H), _f32)]
            + [pltpu.SemaphoreType.DMA for _ in range(5)]
        ),
    )


def _sc_agg(msg_tab, idxc, tvec):
    return _sc_agg_call()(msg_tab, idxc, tvec)


def kernel(x, edge_index, params):
    p = params
    xp = jnp.pad(x, ((0, NP - N), (0, 0)))
    pad = jnp.full((EP - E,), N, jnp.int32)
    srcp = jnp.concatenate([edge_index[0], pad])
    dstp = jnp.concatenate([edge_index[1], pad])
    srcs = jnp.stack([srcp * 2, srcp * 2 + 1]).reshape(NCORE, WP, WIN)
    dst2 = jnp.broadcast_to(dstp.reshape(1, WP, WIN), (NCORE, WP, WIN))
    idxc = jnp.stack([srcs, dst2], axis=2)

    def r2(a):
        return a.reshape(1, -1)

    tv = [jnp.broadcast_to(p['conv%d' % i]['t'].reshape(1), (16,))
          for i in range(3)]
    c0, c1, c2 = p['conv0'], p['conv1'], p['conv2']

    h0, m0 = _fc_call(xp, p['fc_W'], r2(p['fc_b']))
    agg = _sc_agg(m0.reshape(2 * NP, H // 2), idxc, tv[0])
    h1, m1 = _conv0_call(agg, agg, h0, c0['W1'], r2(c0['b1']),
                         r2(c0['g1']), r2(c0['be1']), c0['W2'], r2(c0['b2']))
    agg = _sc_agg(m1.reshape(2 * NP, H // 2), idxc, tv[1])
    h2, m2 = _convmid_call(agg, agg, h1, c1['W1'], r2(c1['b1']),
                           r2(c1['g1']), r2(c1['be1']), c1['W2'],
                           r2(c1['b2']), r2(p['norm1_g']), r2(p['norm1_b']))
    agg = _sc_agg(m2.reshape(2 * NP, H // 2), idxc, tv[2])
    h3 = _convlast_call(agg, agg, h2, c2['W1'], r2(c2['b1']), r2(c2['g1']),
                        r2(c2['be1']), c2['W2'], r2(c2['b2']),
                        r2(p['norm2_g']), r2(p['norm2_b']))
    hp, attn = _attn_call(p['attn_c_b'].reshape(1, 1), h0, h1, h2, h3,
                          p['phi_W'], r2(p['phi_b']),
                          p['attn_a_W'], r2(p['attn_a_b']),
                          p['attn_b_W'], r2(p['attn_b_b']),
                          p['attn_c_W'].reshape(1, -1))
    return _pool_call(attn, hp, p['rho_W'], r2(p['rho_b']),
                      p['cls_W'], r2(p['cls_b']))


# speedup vs baseline: 1.0154x; 1.0154x over previous
import functools

import jax
import jax.numpy as jnp
from jax import lax
from jax.experimental import pallas as pl
from jax.experimental.pallas import tpu as pltpu
from jax.experimental.pallas import tpu_sc as plsc

N = 10000
NP = 10240
H = 128
E = 320000
WIN = 128
WP = 2560
EP = WP * WIN
NCORE = 2
NSUB = 16
WPS = WP // NSUB
KW = 8
RPS = NP // NSUB
B = 1024
G = NP // B

_f32 = jnp.float32


def _dot(a, b):
    return jnp.dot(a, b, preferred_element_type=_f32,
                   precision=lax.Precision.HIGHEST)


def _ln(u, g, b):
    mu = jnp.mean(u, axis=-1, keepdims=True)
    var = jnp.mean((u - mu) ** 2, axis=-1, keepdims=True)
    return (u - mu) / jnp.sqrt(var + 1e-5) * g + b


def _split_agg(a0, a1):
    acc = jnp.concatenate([a0[:, :64], a1[:, :64]], axis=1)
    den = jnp.concatenate([a0[:, 64:], a1[:, 64:]], axis=1)
    return acc, den


def _fc_body(x_ref, w_ref, b_ref, h_ref, m_ref):
    h = jnp.maximum(_dot(x_ref[...], w_ref[...]) + b_ref[...], 0.0)
    h_ref[...] = h
    m_ref[...] = h + 1e-7


def _conv0_body(acc_ref, den_ref, h_ref, w1_ref, b1_ref, g1_ref,
                be1_ref, w2_ref, b2_ref, ho_ref, m_ref):
    acc, den = _split_agg(acc_ref[0], den_ref[0])
    agg = acc / (den + 1e-16) + h_ref[...]
    u = _dot(agg, w1_ref[...]) + b1_ref[...]
    u = jnp.maximum(_ln(u, g1_ref[...], be1_ref[...]), 0.0)
    h1 = _dot(u, w2_ref[...]) + b2_ref[...]
    ho_ref[...] = h1
    m_ref[...] = jnp.maximum(h1, 0.0) + 1e-7


def _convmid_body(acc_ref, den_ref, h_ref, w1_ref, b1_ref, g1_ref,
                  be1_ref, w2_ref, b2_ref, ng_ref, nb_ref, ho_ref, m_ref):
    acc, den = _split_agg(acc_ref[0], den_ref[0])
    agg = acc / (den + 1e-16) + h_ref[...]
    u = _dot(agg, w1_ref[...]) + b1_ref[...]
    u = jnp.maximum(_ln(u, g1_ref[...], be1_ref[...]), 0.0)
    c = _dot(u, w2_ref[...]) + b2_ref[...]
    c = jnp.maximum(_ln(c, ng_ref[...], nb_ref[...]), 0.0)
    hn = h_ref[...] + c
    ho_ref[...] = hn
    m_ref[...] = jnp.maximum(hn, 0.0) + 1e-7


def _convlast_body(acc_ref, den_ref, h_ref, w1_ref, b1_ref, g1_ref,
                   be1_ref, w2_ref, b2_ref, ng_ref, nb_ref, ho_ref):
    acc, den = _split_agg(acc_ref[0], den_ref[0])
    agg = acc / (den + 1e-16) + h_ref[...]
    u = _dot(agg, w1_ref[...]) + b1_ref[...]
    u = jnp.maximum(_ln(u, g1_ref[...], be1_ref[...]), 0.0)
    c = _dot(u, w2_ref[...]) + b2_ref[...]
    c = jnp.maximum(_ln(c, ng_ref[...], nb_ref[...]), 0.0)
    ho_ref[...] = h_ref[...] + c


def _attn_body(cb_ref, h0_ref, h1_ref, h2_ref, h3_ref, pw_ref, pb_ref,
               aw_ref, ab_ref, bw_ref, bb_ref, cw_ref, hp_ref, a_ref):
    hcat = jnp.concatenate(
        [h0_ref[...], h1_ref[...], h2_ref[...], h3_ref[...]], axis=1)
    hp = jnp.maximum(_dot(hcat, pw_ref[...]) + pb_ref[...], 0.0)
    a = jnp.tanh(_dot(hp, aw_ref[...]) + ab_ref[...])
    g = 1.0 / (1.0 + jnp.exp(-(_dot(hp, bw_ref[...]) + bb_ref[...])))
    hp_ref[...] = hp
    a_ref[...] = jnp.sum(a * g * cw_ref[...], axis=1,
                         keepdims=True) + cb_ref[0, 0]


def _pool_body(a_ref, hp_ref, rw_ref, rb_ref, cw_ref, cb_ref, o_ref):
    mask = lax.broadcasted_iota(jnp.int32, (NP, 1), 0) < N
    am = jnp.where(mask, a_ref[...], -jnp.inf)
    mx = jnp.max(am)
    e = jnp.exp(am - mx)
    w = e / jnp.sum(e)
    hp = jnp.where(mask, hp_ref[...], 0.0)
    pooled = jnp.sum(w * hp, axis=0, keepdims=True)
    hr = jnp.maximum(_dot(pooled, rw_ref[...]) + rb_ref[...], 0.0)
    logits = _dot(hr, cw_ref[...]) + cb_ref[...]
    o_ref[...] = 1.0 / (1.0 + jnp.exp(-logits))


def _row(x):
    return pl.BlockSpec((B, x), lambda i: (i, 0))


def _w(shape):
    return pl.BlockSpec(shape, lambda i: (0, 0))


_SM = pl.BlockSpec(memory_space=pltpu.SMEM)

_fc_call = pl.pallas_call(
    _fc_body, grid=(G,),
    in_specs=[_row(H), _w((H, H)), _w((1, H))],
    out_specs=[_row(H), _row(H)],
    out_shape=[jax.ShapeDtypeStruct((NP, H), _f32),
               jax.ShapeDtypeStruct((NP, H), _f32)],
)

_agg0 = pl.BlockSpec((1, B, H), lambda i: (0, i, 0))
_agg1 = pl.BlockSpec((1, B, H), lambda i: (1, i, 0))

_conv0_call = pl.pallas_call(
    _conv0_body, grid=(G,),
    in_specs=[_agg0, _agg1, _row(H), _w((H, 2 * H)), _w((1, 2 * H)),
              _w((1, 2 * H)), _w((1, 2 * H)), _w((2 * H, H)), _w((1, H))],
    out_specs=[_row(H), _row(H)],
    out_shape=[jax.ShapeDtypeStruct((NP, H), _f32),
               jax.ShapeDtypeStruct((NP, H), _f32)],
)

_convmid_call = pl.pallas_call(
    _convmid_body, grid=(G,),
    in_specs=[_agg0, _agg1, _row(H), _w((H, 2 * H)), _w((1, 2 * H)),
              _w((1, 2 * H)), _w((1, 2 * H)), _w((2 * H, H)), _w((1, H)),
              _w((1, H)), _w((1, H))],
    out_specs=[_row(H), _row(H)],
    out_shape=[jax.ShapeDtypeStruct((NP, H), _f32),
               jax.ShapeDtypeStruct((NP, H), _f32)],
)

_convlast_call = pl.pallas_call(
    _convlast_body, grid=(G,),
    in_specs=[_agg0, _agg1, _row(H), _w((H, 2 * H)), _w((1, 2 * H)),
              _w((1, 2 * H)), _w((1, 2 * H)), _w((2 * H, H)), _w((1, H)),
              _w((1, H)), _w((1, H))],
    out_specs=_row(H),
    out_shape=jax.ShapeDtypeStruct((NP, H), _f32),
)

_attn_call = pl.pallas_call(
    _attn_body, grid=(G,),
    in_specs=[_SM, _row(H), _row(H), _row(H), _row(H),
              _w((4 * H, 4 * H)), _w((1, 4 * H)),
              _w((4 * H, 4 * H)), _w((1, 4 * H)),
              _w((4 * H, 4 * H)), _w((1, 4 * H)), _w((1, 4 * H))],
    out_specs=[_row(4 * H), pl.BlockSpec((B, 1), lambda i: (i, 0))],
    out_shape=[jax.ShapeDtypeStruct((NP, 4 * H), _f32),
               jax.ShapeDtypeStruct((NP, 1), _f32)],
)

_pool_call = pl.pallas_call(
    _pool_body,
    in_specs=[pl.BlockSpec((NP, 1), lambda: (0, 0)),
              pl.BlockSpec((NP, 4 * H), lambda: (0, 0)),
              pl.BlockSpec((4 * H, 4 * H), lambda: (0, 0)),
              pl.BlockSpec((1, 4 * H), lambda: (0, 0)),
              pl.BlockSpec((4 * H, 2), lambda: (0, 0)),
              pl.BlockSpec((1, 2), lambda: (0, 0))],
    out_specs=pl.BlockSpec((1, 2), lambda: (0, 0)),
    out_shape=jax.ShapeDtypeStruct((1, 2), _f32),
)


KG = 16
NCHUNK = WPS // KG
GPC = KG // 2


def _sc_agg_body(msg_ref, idxc_ref, tv_ref, out_ref,
                 mb0, mb1, sbuf, ib0, ib1, tvm, spacc,
                 g0, g1, s0, i0, i1):
    c = lax.axis_index("c")
    s = lax.axis_index("s")

    @pl.loop(0, WIN)
    def _zrow(i):
        @pl.loop(0, H // 16)
        def _zcol(j):
            sbuf[i, pl.ds(j * 16, 16)] = jnp.zeros((16,), _f32)

    @pl.loop(0, RPS // WIN)
    def _zacc(k):
        pltpu.sync_copy(sbuf, spacc.at[pl.ds(s * RPS + k * WIN, WIN)])

    pltpu.sync_copy(tv_ref, tvm)
    tv = tvm[...]

    def idx_load(cc, ib, isem):
        pltpu.async_copy(
            idxc_ref.at[c, pl.ds(s * WPS + cc * KG, KG)], ib, isem)

    def compute(mb):
        @plsc.parallel_loop(0, WIN, unroll=8)
        def _e(e):
            for j in range(4):
                m = mb[e, pl.ds(j * 16, 16)]
                q = jnp.exp(m * tv)
                sbuf[e, pl.ds(64 + j * 16, 16)] = q
                sbuf[e, pl.ds(j * 16, 16)] = m * q

    def do_chunk(ib):
        pltpu.async_copy(msg_ref.at[ib.at[0, 0]], mb0, g0)
        pltpu.async_copy(msg_ref.at[ib.at[1, 0]], mb1, g1)

        @pl.loop(0, GPC)
        def _pair(gg):
            w = gg * 2
            pltpu.make_async_copy(msg_ref.at[ib.at[w, 0]], mb0, g0).wait()
            compute(mb0)
            pltpu.async_copy(sbuf, spacc.at[ib.at[w, 1]], s0, add=True)

            @pl.when(gg < GPC - 1)
            def _f0():
                pltpu.async_copy(msg_ref.at[ib.at[w + 2, 0]], mb0, g0)

            pltpu.make_async_copy(msg_ref.at[ib.at[w + 1, 0]], mb1, g1).wait()
            pltpu.make_async_copy(sbuf, spacc.at[ib.at[w, 1]], s0).wait()
            compute(mb1)
            pltpu.async_copy(sbuf, spacc.at[ib.at[w + 1, 1]], s0, add=True)

            @pl.when(gg < GPC - 1)
            def _f1():
                pltpu.async_copy(msg_ref.at[ib.at[w + 3, 0]], mb1, g1)

            pltpu.make_async_copy(sbuf, spacc.at[ib.at[w + 1, 1]], s0).wait()

    idx_load(0, ib0, i0)
    plsc.subcore_barrier()

    @pl.loop(0, NCHUNK // 2)
    def _chunkpair(k):
        cc = k * 2
        pltpu.make_async_copy(
            idxc_ref.at[c, pl.ds(s * WPS + cc * KG, KG)], ib0, i0).wait()
        idx_load(cc + 1, ib1, i1)
        do_chunk(ib0)
        pltpu.make_async_copy(
            idxc_ref.at[c, pl.ds(s * WPS + (cc + 1) * KG, KG)], ib1, i1).wait()

        @pl.when(k < NCHUNK // 2 - 1)
        def _nxt():
            idx_load(cc + 2, ib0, i0)

        do_chunk(ib1)

    plsc.subcore_barrier()
    pltpu.sync_copy(spacc.at[pl.ds(s * RPS, RPS)],
                    out_ref.at[c, pl.ds(s * RPS, RPS)])


@functools.cache
def _sc_agg_call():
    return pl.kernel(
        _sc_agg_body,
        out_type=jax.ShapeDtypeStruct((NCORE, NP, H), _f32),
        mesh=plsc.VectorSubcoreMesh(core_axis_name="c", subcore_axis_name="s"),
        compiler_params=pltpu.CompilerParams(use_tc_tiling_on_sc=False),
        scratch_types=(
            [pltpu.VMEM((WIN, H // 2), _f32) for _ in range(2)]
            + [pltpu.VMEM((WIN, H), _f32)]
            + [pltpu.VMEM((KG, 2, WIN), jnp.int32) for _ in range(2)]
            + [pltpu.VMEM((16,), _f32)]
            + [pltpu.VMEM_SHARED((NP, H), _f32)]
            + [pltpu.SemaphoreType.DMA for _ in range(5)]
        ),
    )


def _sc_agg(msg_tab, idxc, tvec):
    return _sc_agg_call()(msg_tab, idxc, tvec)


def kernel(x, edge_index, params):
    p = params
    xp = jnp.pad(x, ((0, NP - N), (0, 0)))
    pad = jnp.full((EP - E,), N, jnp.int32)
    srcp = jnp.concatenate([edge_index[0], pad])
    dstp = jnp.concatenate([edge_index[1], pad])
    srcs = jnp.stack([srcp * 2, srcp * 2 + 1]).reshape(NCORE, WP, WIN)
    dst2 = jnp.broadcast_to(dstp.reshape(1, WP, WIN), (NCORE, WP, WIN))
    idxc = jnp.stack([srcs, dst2], axis=2)

    def r2(a):
        return a.reshape(1, -1)

    tv = [jnp.broadcast_to(p['conv%d' % i]['t'].reshape(1), (16,))
          for i in range(3)]
    c0, c1, c2 = p['conv0'], p['conv1'], p['conv2']

    h0, m0 = _fc_call(xp, p['fc_W'], r2(p['fc_b']))
    agg = _sc_agg(m0.reshape(2 * NP, H // 2), idxc, tv[0])
    h1, m1 = _conv0_call(agg, agg, h0, c0['W1'], r2(c0['b1']),
                         r2(c0['g1']), r2(c0['be1']), c0['W2'], r2(c0['b2']))
    agg = _sc_agg(m1.reshape(2 * NP, H // 2), idxc, tv[1])
    h2, m2 = _convmid_call(agg, agg, h1, c1['W1'], r2(c1['b1']),
                           r2(c1['g1']), r2(c1['be1']), c1['W2'],
                           r2(c1['b2']), r2(p['norm1_g']), r2(p['norm1_b']))
    agg = _sc_agg(m2.reshape(2 * NP, H // 2), idxc, tv[2])
    h3 = _convlast_call(agg, agg, h2, c2['W1'], r2(c2['b1']), r2(c2['g1']),
                        r2(c2['be1']), c2['W2'], r2(c2['b2']),
                        r2(p['norm2_g']), r2(p['norm2_b']))
    hp, attn = _attn_call(p['attn_c_b'].reshape(1, 1), h0, h1, h2, h3,
                          p['phi_W'], r2(p['phi_b']),
                          p['attn_a_W'], r2(p['attn_a_b']),
                          p['attn_b_W'], r2(p['attn_b_b']),
                          p['attn_c_W'].reshape(1, -1))
    return _pool_call(attn, hp, p['rho_W'], r2(p['rho_b']),
                      p['cls_W'], r2(p['cls_b']))


# speedup vs baseline: 1.0167x; 1.0014x over previous
import functools

import jax
import jax.numpy as jnp
from jax import lax
from jax.experimental import pallas as pl
from jax.experimental.pallas import tpu as pltpu
from jax.experimental.pallas import tpu_sc as plsc

N = 10000
NP = 10240
H = 128
E = 320000
WIN = 128
WP = 2560
EP = WP * WIN
NCORE = 2
NSUB = 16
WPS = WP // NSUB
KW = 8
RPS = NP // NSUB
B = 1024
G = NP // B

_f32 = jnp.float32


def _dot(a, b):
    return jnp.dot(a, b, preferred_element_type=_f32,
                   precision=lax.Precision.HIGHEST)


def _ln(u, g, b):
    mu = jnp.mean(u, axis=-1, keepdims=True)
    var = jnp.mean((u - mu) ** 2, axis=-1, keepdims=True)
    return (u - mu) / jnp.sqrt(var + 1e-5) * g + b


def _split_agg(a0, a1):
    acc = jnp.concatenate([a0[:, :64], a1[:, :64]], axis=1)
    den = jnp.concatenate([a0[:, 64:], a1[:, 64:]], axis=1)
    return acc, den


def _fc_body(x_ref, w_ref, b_ref, h_ref, m_ref):
    h = jnp.maximum(_dot(x_ref[...], w_ref[...]) + b_ref[...], 0.0)
    h_ref[...] = h
    m_ref[...] = h + 1e-7


def _conv0_body(acc_ref, den_ref, h_ref, w1_ref, b1_ref, g1_ref,
                be1_ref, w2_ref, b2_ref, ho_ref, m_ref):
    acc, den = _split_agg(acc_ref[0], den_ref[0])
    agg = acc / (den + 1e-16) + h_ref[...]
    u = _dot(agg, w1_ref[...]) + b1_ref[...]
    u = jnp.maximum(_ln(u, g1_ref[...], be1_ref[...]), 0.0)
    h1 = _dot(u, w2_ref[...]) + b2_ref[...]
    ho_ref[...] = h1
    m_ref[...] = jnp.maximum(h1, 0.0) + 1e-7


def _convmid_body(acc_ref, den_ref, h_ref, w1_ref, b1_ref, g1_ref,
                  be1_ref, w2_ref, b2_ref, ng_ref, nb_ref, ho_ref, m_ref):
    acc, den = _split_agg(acc_ref[0], den_ref[0])
    agg = acc / (den + 1e-16) + h_ref[...]
    u = _dot(agg, w1_ref[...]) + b1_ref[...]
    u = jnp.maximum(_ln(u, g1_ref[...], be1_ref[...]), 0.0)
    c = _dot(u, w2_ref[...]) + b2_ref[...]
    c = jnp.maximum(_ln(c, ng_ref[...], nb_ref[...]), 0.0)
    hn = h_ref[...] + c
    ho_ref[...] = hn
    m_ref[...] = jnp.maximum(hn, 0.0) + 1e-7


def _convlast_body(acc_ref, den_ref, h_ref, w1_ref, b1_ref, g1_ref,
                   be1_ref, w2_ref, b2_ref, ng_ref, nb_ref, ho_ref):
    acc, den = _split_agg(acc_ref[0], den_ref[0])
    agg = acc / (den + 1e-16) + h_ref[...]
    u = _dot(agg, w1_ref[...]) + b1_ref[...]
    u = jnp.maximum(_ln(u, g1_ref[...], be1_ref[...]), 0.0)
    c = _dot(u, w2_ref[...]) + b2_ref[...]
    c = jnp.maximum(_ln(c, ng_ref[...], nb_ref[...]), 0.0)
    ho_ref[...] = h_ref[...] + c


def _attn_body(cb_ref, h0_ref, h1_ref, h2_ref, h3_ref, pw_ref, pb_ref,
               aw_ref, ab_ref, bw_ref, bb_ref, cw_ref, hp_ref, a_ref):
    hcat = jnp.concatenate(
        [h0_ref[...], h1_ref[...], h2_ref[...], h3_ref[...]], axis=1)
    hp = jnp.maximum(_dot(hcat, pw_ref[...]) + pb_ref[...], 0.0)
    a = jnp.tanh(_dot(hp, aw_ref[...]) + ab_ref[...])
    g = 1.0 / (1.0 + jnp.exp(-(_dot(hp, bw_ref[...]) + bb_ref[...])))
    hp_ref[...] = hp
    a_ref[...] = jnp.sum(a * g * cw_ref[...], axis=1,
                         keepdims=True) + cb_ref[0, 0]


def _pool_body(a_ref, hp_ref, rw_ref, rb_ref, cw_ref, cb_ref, o_ref):
    mask = lax.broadcasted_iota(jnp.int32, (NP, 1), 0) < N
    am = jnp.where(mask, a_ref[...], -jnp.inf)
    mx = jnp.max(am)
    e = jnp.exp(am - mx)
    w = e / jnp.sum(e)
    hp = jnp.where(mask, hp_ref[...], 0.0)
    pooled = jnp.sum(w * hp, axis=0, keepdims=True)
    hr = jnp.maximum(_dot(pooled, rw_ref[...]) + rb_ref[...], 0.0)
    logits = _dot(hr, cw_ref[...]) + cb_ref[...]
    o_ref[...] = 1.0 / (1.0 + jnp.exp(-logits))


def _row(x):
    return pl.BlockSpec((B, x), lambda i: (i, 0))


def _w(shape):
    return pl.BlockSpec(shape, lambda i: (0, 0))


_SM = pl.BlockSpec(memory_space=pltpu.SMEM)

_fc_call = pl.pallas_call(
    _fc_body, grid=(G,),
    in_specs=[_row(H), _w((H, H)), _w((1, H))],
    out_specs=[_row(H), _row(H)],
    out_shape=[jax.ShapeDtypeStruct((NP, H), _f32),
               jax.ShapeDtypeStruct((NP, H), _f32)],
)

_agg0 = pl.BlockSpec((1, B, H), lambda i: (0, i, 0))
_agg1 = pl.BlockSpec((1, B, H), lambda i: (1, i, 0))

_conv0_call = pl.pallas_call(
    _conv0_body, grid=(G,),
    in_specs=[_agg0, _agg1, _row(H), _w((H, 2 * H)), _w((1, 2 * H)),
              _w((1, 2 * H)), _w((1, 2 * H)), _w((2 * H, H)), _w((1, H))],
    out_specs=[_row(H), _row(H)],
    out_shape=[jax.ShapeDtypeStruct((NP, H), _f32),
               jax.ShapeDtypeStruct((NP, H), _f32)],
)

_convmid_call = pl.pallas_call(
    _convmid_body, grid=(G,),
    in_specs=[_agg0, _agg1, _row(H), _w((H, 2 * H)), _w((1, 2 * H)),
              _w((1, 2 * H)), _w((1, 2 * H)), _w((2 * H, H)), _w((1, H)),
              _w((1, H)), _w((1, H))],
    out_specs=[_row(H), _row(H)],
    out_shape=[jax.ShapeDtypeStruct((NP, H), _f32),
               jax.ShapeDtypeStruct((NP, H), _f32)],
)

_convlast_call = pl.pallas_call(
    _convlast_body, grid=(G,),
    in_specs=[_agg0, _agg1, _row(H), _w((H, 2 * H)), _w((1, 2 * H)),
              _w((1, 2 * H)), _w((1, 2 * H)), _w((2 * H, H)), _w((1, H)),
              _w((1, H)), _w((1, H))],
    out_specs=_row(H),
    out_shape=jax.ShapeDtypeStruct((NP, H), _f32),
)

_attn_call = pl.pallas_call(
    _attn_body, grid=(G,),
    in_specs=[_SM, _row(H), _row(H), _row(H), _row(H),
              _w((4 * H, 4 * H)), _w((1, 4 * H)),
              _w((4 * H, 4 * H)), _w((1, 4 * H)),
              _w((4 * H, 4 * H)), _w((1, 4 * H)), _w((1, 4 * H))],
    out_specs=[_row(4 * H), pl.BlockSpec((B, 1), lambda i: (i, 0))],
    out_shape=[jax.ShapeDtypeStruct((NP, 4 * H), _f32),
               jax.ShapeDtypeStruct((NP, 1), _f32)],
)

_pool_call = pl.pallas_call(
    _pool_body,
    in_specs=[pl.BlockSpec((NP, 1), lambda: (0, 0)),
              pl.BlockSpec((NP, 4 * H), lambda: (0, 0)),
              pl.BlockSpec((4 * H, 4 * H), lambda: (0, 0)),
              pl.BlockSpec((1, 4 * H), lambda: (0, 0)),
              pl.BlockSpec((4 * H, 2), lambda: (0, 0)),
              pl.BlockSpec((1, 2), lambda: (0, 0))],
    out_specs=pl.BlockSpec((1, 2), lambda: (0, 0)),
    out_shape=jax.ShapeDtypeStruct((1, 2), _f32),
)


KG = 20
NCHUNK = WPS // KG
GPC = KG // 2


def _sc_agg_body(msg_ref, idxc_ref, tv_ref, out_ref,
                 mb0, mb1, sbuf, ib0, ib1, tvm, spacc,
                 g0, g1, s0, i0, i1):
    c = lax.axis_index("c")
    s = lax.axis_index("s")

    @pl.loop(0, WIN)
    def _zrow(i):
        @pl.loop(0, H // 16)
        def _zcol(j):
            sbuf[i, pl.ds(j * 16, 16)] = jnp.zeros((16,), _f32)

    @pl.loop(0, RPS // WIN)
    def _zacc(k):
        pltpu.sync_copy(sbuf, spacc.at[pl.ds(s * RPS + k * WIN, WIN)])

    pltpu.sync_copy(tv_ref, tvm)
    tv = tvm[...]

    def idx_load(cc, ib, isem):
        pltpu.async_copy(
            idxc_ref.at[c, pl.ds(s * WPS + cc * KG, KG)], ib, isem)

    def compute(mb):
        @plsc.parallel_loop(0, WIN, unroll=8)
        def _e(e):
            for j in range(4):
                m = mb[e, pl.ds(j * 16, 16)]
                q = jnp.exp(m * tv)
                sbuf[e, pl.ds(64 + j * 16, 16)] = q
                sbuf[e, pl.ds(j * 16, 16)] = m * q

    def do_chunk(ib):
        pltpu.async_copy(msg_ref.at[ib.at[0, 0]], mb0, g0)
        pltpu.async_copy(msg_ref.at[ib.at[1, 0]], mb1, g1)

        @pl.loop(0, GPC)
        def _pair(gg):
            w = gg * 2
            pltpu.make_async_copy(msg_ref.at[ib.at[w, 0]], mb0, g0).wait()
            compute(mb0)
            pltpu.async_copy(sbuf, spacc.at[ib.at[w, 1]], s0, add=True)

            @pl.when(gg < GPC - 1)
            def _f0():
                pltpu.async_copy(msg_ref.at[ib.at[w + 2, 0]], mb0, g0)

            pltpu.make_async_copy(msg_ref.at[ib.at[w + 1, 0]], mb1, g1).wait()
            pltpu.make_async_copy(sbuf, spacc.at[ib.at[w, 1]], s0).wait()
            compute(mb1)
            pltpu.async_copy(sbuf, spacc.at[ib.at[w + 1, 1]], s0, add=True)

            @pl.when(gg < GPC - 1)
            def _f1():
                pltpu.async_copy(msg_ref.at[ib.at[w + 3, 0]], mb1, g1)

            pltpu.make_async_copy(sbuf, spacc.at[ib.at[w + 1, 1]], s0).wait()

    idx_load(0, ib0, i0)
    plsc.subcore_barrier()

    @pl.loop(0, NCHUNK // 2)
    def _chunkpair(k):
        cc = k * 2
        pltpu.make_async_copy(
            idxc_ref.at[c, pl.ds(s * WPS + cc * KG, KG)], ib0, i0).wait()
        idx_load(cc + 1, ib1, i1)
        do_chunk(ib0)
        pltpu.make_async_copy(
            idxc_ref.at[c, pl.ds(s * WPS + (cc + 1) * KG, KG)], ib1, i1).wait()

        @pl.when(k < NCHUNK // 2 - 1)
        def _nxt():
            idx_load(cc + 2, ib0, i0)

        do_chunk(ib1)

    plsc.subcore_barrier()
    pltpu.sync_copy(spacc.at[pl.ds(s * RPS, RPS)],
                    out_ref.at[c, pl.ds(s * RPS, RPS)])


@functools.cache
def _sc_agg_call():
    return pl.kernel(
        _sc_agg_body,
        out_type=jax.ShapeDtypeStruct((NCORE, NP, H), _f32),
        mesh=plsc.VectorSubcoreMesh(core_axis_name="c", subcore_axis_name="s"),
        compiler_params=pltpu.CompilerParams(use_tc_tiling_on_sc=False),
        scratch_types=(
            [pltpu.VMEM((WIN, H // 2), _f32) for _ in range(2)]
            + [pltpu.VMEM((WIN, H), _f32)]
            + [pltpu.VMEM((KG, 2, WIN), jnp.int32) for _ in range(2)]
            + [pltpu.VMEM((16,), _f32)]
            + [pltpu.VMEM_SHARED((NP, H), _f32)]
            + [pltpu.SemaphoreType.DMA for _ in range(5)]
        ),
    )


def _sc_agg(msg_tab, idxc, tvec):
    return _sc_agg_call()(msg_tab, idxc, tvec)


def kernel(x, edge_index, params):
    p = params
    xp = jnp.pad(x, ((0, NP - N), (0, 0)))
    pad = jnp.full((EP - E,), N, jnp.int32)
    srcp = jnp.concatenate([edge_index[0], pad])
    dstp = jnp.concatenate([edge_index[1], pad])
    srcs = jnp.stack([srcp * 2, srcp * 2 + 1]).reshape(NCORE, WP, WIN)
    dst2 = jnp.broadcast_to(dstp.reshape(1, WP, WIN), (NCORE, WP, WIN))
    idxc = jnp.stack([srcs, dst2], axis=2)

    def r2(a):
        return a.reshape(1, -1)

    tv = [jnp.broadcast_to(p['conv%d' % i]['t'].reshape(1), (16,))
          for i in range(3)]
    c0, c1, c2 = p['conv0'], p['conv1'], p['conv2']

    h0, m0 = _fc_call(xp, p['fc_W'], r2(p['fc_b']))
    agg = _sc_agg(m0.reshape(2 * NP, H // 2), idxc, tv[0])
    h1, m1 = _conv0_call(agg, agg, h0, c0['W1'], r2(c0['b1']),
                         r2(c0['g1']), r2(c0['be1']), c0['W2'], r2(c0['b2']))
    agg = _sc_agg(m1.reshape(2 * NP, H // 2), idxc, tv[1])
    h2, m2 = _convmid_call(agg, agg, h1, c1['W1'], r2(c1['b1']),
                           r2(c1['g1']), r2(c1['be1']), c1['W2'],
                           r2(c1['b2']), r2(p['norm1_g']), r2(p['norm1_b']))
    agg = _sc_agg(m2.reshape(2 * NP, H // 2), idxc, tv[2])
    h3 = _convlast_call(agg, agg, h2, c2['W1'], r2(c2['b1']), r2(c2['g1']),
                        r2(c2['be1']), c2['W2'], r2(c2['b2']),
                        r2(p['norm2_g']), r2(p['norm2_b']))
    hp, attn = _attn_call(p['attn_c_b'].reshape(1, 1), h0, h1, h2, h3,
                          p['phi_W'], r2(p['phi_b']),
                          p['attn_a_W'], r2(p['attn_a_b']),
                          p['attn_b_W'], r2(p['attn_b_b']),
                          p['attn_c_W'].reshape(1, -1))
    return _pool_call(attn, hp, p['rho_W'], r2(p['rho_b']),
                      p['cls_W'], r2(p['cls_b']))
